# baseline (speedup 1.0000x reference)
import functools
import jax
import jax.numpy as jnp
from jax import lax
from jax.experimental import pallas as pl
from jax.experimental.pallas import tpu as pltpu
from jax.experimental.pallas import tpu_sc as plsc

DEPTH = 3
N_MOLS = 500
N_FRAG_MOLS = 500
BN = 512
NC, NS = 2, 16
NW = NC * NS


def _cdivmul(n, m):
    return -(-n // m) * m


def tc_matmul(x, w, b=None, add=None, addcat=None, relu=False, both=False):
    n, k = x.shape
    h = w.shape[1]
    npad = -n % BN
    if npad:
        x = jnp.pad(x, ((0, npad), (0, 0)))
        if add is not None:
            add = jnp.pad(add, ((0, npad), (0, 0)))
    np_ = x.shape[0]
    grid = (np_ // BN,)
    xspec = pl.BlockSpec((BN, k), lambda i: (i, 0))
    wspec = pl.BlockSpec((k, h), lambda i: (0, 0))
    bspec = pl.BlockSpec((1, h), lambda i: (0, 0))
    ospec = pl.BlockSpec((BN, h), lambda i: (i, 0))
    in_specs = [xspec, wspec]
    args = [x, w]
    if b is not None:
        in_specs.append(bspec)
        args.append(b.reshape(1, h))
    if add is not None:
        in_specs.append(ospec)
        args.append(add)
    if addcat is not None:
        acat, tcol = addcat
        in_specs.append(pl.BlockSpec((BN, h), lambda i, _t=tcol: (i, _t)))
        args.append(acat)
    if both:
        out_shape = (jax.ShapeDtypeStruct((np_, h), jnp.float32),) * 2
        out_specs = (ospec, ospec)
    else:
        out_shape = jax.ShapeDtypeStruct((np_, h), jnp.float32)
        out_specs = ospec

    def body(*refs):
        it = iter(refs)
        x_ref = next(it)
        w_ref = next(it)
        b_ref = next(it) if b is not None else None
        a_ref = next(it) if add is not None else None
        ac_ref = next(it) if addcat is not None else None
        acc = jnp.dot(x_ref[...], w_ref[...], preferred_element_type=jnp.float32)
        if b_ref is not None:
            acc = acc + b_ref[...]
        if a_ref is not None:
            acc = acc + a_ref[...]
        if ac_ref is not None:
            acc = acc + ac_ref[...]
        if both:
            next(it)[...] = acc
            next(it)[...] = jnp.maximum(acc, 0.0)
        elif relu:
            next(it)[...] = jnp.maximum(acc, 0.0)
        else:
            next(it)[...] = acc

    out = pl.pallas_call(
        body, grid=grid, in_specs=in_specs, out_specs=out_specs,
        out_shape=out_shape,
        compiler_params=pltpu.CompilerParams(
            dimension_semantics=("parallel",)))(*args)
    if both:
        return (out[0][:n], out[1][:n]) if npad else out
    return out[:n] if npad else out


def _pairswap(m):
    up = jnp.concatenate([m[1:], m[:1]], axis=0)
    dn = jnp.concatenate([m[-1:], m[:-1]], axis=0)
    rows = lax.broadcasted_iota(jnp.int32, m.shape, 0)
    return jnp.where(rows % 2 == 0, up, dn)


def tc_combine(x, w, inp, g1):
    n, h = x.shape
    assert n % BN == 0
    grid = (n // BN,)
    spec = pl.BlockSpec((BN, h), lambda i: (i, 0))
    wspec = pl.BlockSpec((h, h), lambda i: (0, 0))

    def body(x_ref, w_ref, inp_ref, g1_ref, o_ref):
        m = jnp.dot(x_ref[...], w_ref[...], preferred_element_type=jnp.float32)
        o_ref[...] = jnp.maximum(inp_ref[...] + g1_ref[...] - _pairswap(m), 0.0)

    return pl.pallas_call(
        body, grid=grid, in_specs=[spec, wspec, spec, spec], out_specs=spec,
        out_shape=jax.ShapeDtypeStruct((n, h), jnp.float32),
        compiler_params=pltpu.CompilerParams(
            dimension_semantics=("parallel",)))(x, w, inp, g1)


def _sc_mesh():
    return plsc.VectorSubcoreMesh(core_axis_name="c", subcore_axis_name="s")


def _wid():
    return lax.axis_index("s") * NC + lax.axis_index("c")


def sc_gather_rows(table, idx):
    v, hc = table.shape
    s = 2 if hc <= 128 else 1
    ch = s * 128
    sc_rows = 1024
    rounds = sc_rows // ch
    b = idx.shape[0]
    bp = _cdivmul(b, NW * sc_rows)
    if bp != b:
        idx = jnp.pad(idx, (0, bp - b))
    idx2 = idx.reshape(bp // 128, 128)
    bpw = bp // NW
    nch = bpw // sc_rows

    @functools.partial(
        pl.kernel,
        out_type=jax.ShapeDtypeStruct((bp, hc), jnp.float32),
        mesh=_sc_mesh(),
        scratch_types=[pltpu.VMEM((8, 128), jnp.int32),
                       pltpu.VMEM((ch, hc), jnp.float32),
                       pltpu.VMEM((ch, hc), jnp.float32),
                       pltpu.SemaphoreType.DMA,
                       pltpu.SemaphoreType.DMA],
    )
    def k(table_h, idx_h, out_h, idx_v, rows_a, rows_b, isem, gsem):
        base = _wid() * bpw

        def idx_cp(i):
            off = pl.multiple_of(base + i * sc_rows, 1024)
            return pltpu.make_async_copy(
                idx_h.at[pl.ds(pl.multiple_of(off // 128, 8), 8)], idx_v, isem)

        def fire(buf, rr):
            return [pltpu.async_copy(table_h.at[idx_v.at[rr * s + j]],
                                     buf.at[pl.ds(j * 128, 128)], gsem)
                    for j in range(s)]

        idx_cp(0).start()

        def chunk(i, c):
            off = pl.multiple_of(base + i * sc_rows, 1024)
            idx_cp(i).wait()
            cps = fire(rows_a, 0)
            for rr in range(rounds):
                buf = rows_a if rr % 2 == 0 else rows_b
                for cp in cps:
                    cp.wait()
                if rr + 1 < rounds:
                    cps = fire(rows_b if rr % 2 == 0 else rows_a, rr + 1)
                pltpu.sync_copy(
                    buf,
                    out_h.at[pl.ds(pl.multiple_of(off + rr * ch, ch), ch)])

            @pl.when(i + 1 < nch)
            def _():
                idx_cp(i + 1).start()

            return c

        lax.fori_loop(0, nch, chunk, 0)

    return k(table, idx2)


def sc_gather_sum(table, idx2d, nap):
    na, nb = idx2d.shape
    v, h = table.shape
    s = 2 if h <= 128 else 1
    ch = s * 128
    arh = ch // nb
    asc = 1024 // nb
    rounds = 1024 // ch
    assert nap % (NW * asc) == 0
    idx = idx2d
    if nap != na:
        idx = jnp.pad(idx, ((0, nap - na), (0, 0)))
    idxf = idx.reshape(nap * nb // 128, 128)
    apw = nap // NW
    nch = apw // asc
    hb = h // 16

    @functools.partial(
        pl.kernel,
        out_type=jax.ShapeDtypeStruct((nap, h), jnp.float32),
        mesh=_sc_mesh(),
        scratch_types=[pltpu.VMEM((8, 128), jnp.int32),
                       pltpu.VMEM((ch, h), jnp.float32),
                       pltpu.VMEM((ch, h), jnp.float32),
                       pltpu.VMEM((asc, h), jnp.float32),
                       pltpu.SemaphoreType.DMA,
                       pltpu.SemaphoreType.DMA],
    )
    def k(table_h, idx_h, out_h, idx_v, rows_a, rows_b, out_v, isem, gsem):
        base = _wid() * apw

        def idx_cp(i):
            aoff = pl.multiple_of(base + i * asc, asc)
            return pltpu.make_async_copy(
                idx_h.at[pl.ds(pl.multiple_of(aoff * nb // 128, 8), 8)],
                idx_v, isem)

        def fire(buf, rr):
            for j in range(s):
                pltpu.make_async_copy(table_h.at[idx_v.at[rr * s + j]],
                                      buf.at[pl.ds(j * 128, 128)],
                                      gsem).start()

        def drain(buf):
            for j in range(s):
                pltpu.make_async_copy(table_h.at[idx_v.at[0]],
                                      buf.at[pl.ds(j * 128, 128)],
                                      gsem).wait()

        def consume(buf, rr):
            def atom(a, c2):
                r0 = a * nb
                for hh in range(hb):
                    sl = pl.ds(hh * 16, 16)
                    acc = buf[r0, sl]
                    for j in range(1, nb):
                        acc = acc + buf[r0 + j, sl]
                    out_v[rr * arh + a, sl] = acc
                return c2

            lax.fori_loop(0, arh, atom, 0)

        idx_cp(0).start()

        def chunk(i, c):
            aoff = pl.multiple_of(base + i * asc, asc)
            idx_cp(i).wait()
            fire(rows_a, 0)

            def rpair(rp, c2):
                rr0 = 2 * rp
                drain(rows_a)
                fire(rows_b, rr0 + 1)
                consume(rows_a, rr0)
                drain(rows_b)

                @pl.when(rr0 + 2 < rounds)
                def _():
                    fire(rows_a, rr0 + 2)

                consume(rows_b, rr0 + 1)
                return c2

            lax.fori_loop(0, rounds // 2, rpair, 0)
            pltpu.sync_copy(out_v,
                            out_h.at[pl.ds(pl.multiple_of(aoff, asc), asc)])

            @pl.when(i + 1 < nch)
            def _():
                idx_cp(i + 1).start()

            return c

        lax.fori_loop(0, nch, chunk, 0)

    return k(table, idxf)


def sc_segsum(x, seg, nsegp):
    npts, h = x.shape
    assert npts % (NW * 128) == 0
    apw = npts // NW
    nch = apw // 128
    zeros = jnp.zeros((nsegp, h), jnp.float32)

    @functools.partial(
        pl.kernel,
        out_type=jax.ShapeDtypeStruct((NC, nsegp, h), jnp.float32),
        mesh=_sc_mesh(),
        scratch_types=[pltpu.VMEM((128,), jnp.int32),
                       pltpu.VMEM((128, h), jnp.float32),
                       pltpu.VMEM_SHARED((nsegp, h), jnp.float32)],
    )
    def k(x_h, seg_h, z_h, out_h, seg_v, x_v, acc_sh):
        sid = lax.axis_index("s")
        cid = lax.axis_index("c")
        base = _wid() * apw

        @pl.when(sid == 0)
        def _():
            pltpu.sync_copy(z_h, acc_sh)

        plsc.subcore_barrier()

        def chunk(i, c):
            off = pl.multiple_of(base + i * 128, 128)
            pltpu.sync_copy(seg_h.at[pl.ds(off, 128)], seg_v)
            pltpu.sync_copy(x_h.at[pl.ds(off, 128)], x_v)
            pltpu.sync_copy(x_v, acc_sh.at[seg_v], add=True)
            return c

        lax.fori_loop(0, nch, chunk, 0)
        plsc.subcore_barrier()

        @pl.when(sid == 0)
        def _():
            pltpu.sync_copy(acc_sh, out_h.at[cid])

    return k(x, seg, zeros)


def _segmean(x, seg, n, npad, nsegp, counts):
    xp = jnp.pad(x, ((0, npad - x.shape[0]), (0, 0)))
    sp = jnp.pad(seg.astype(jnp.int32), (0, npad - seg.shape[0]))
    parts = sc_segsum(xp, sp, nsegp)
    sums = parts[0, :n] + parts[1, :n]
    return jnp.where(counts[:, None] > 0,
                     sums / jnp.maximum(counts, 1.0)[:, None], 0.0)


@jax.jit
def _run(f_atoms, f_bonds, a2b, b2a, b2revb, atom_seg,
         f_frags_atoms, f_frags_bonds, frags_a2b, frags_b2a, frags_b2revb,
         frags_atom_seg, a2frag, W_i, W_h, W_fusion, b_fusion, W_o, b_o):
    H = W_h.shape[0]
    NA, MAXNB = a2b.shape
    NB_ = b2a.shape[0]
    FNA, FMAXNB = frags_a2b.shape
    NAP = _cdivmul(NA, 1024)
    FNAP = _cdivmul(FNA, 2048)
    NSP = _cdivmul(NA, NW * 128)
    FNSP = _cdivmul(FNA, NW * 128)
    NSEGP = _cdivmul(N_MOLS, 8)

    Wf1, Wf2 = W_fusion[:H], W_fusion[H:]

    a2b = a2b.astype(jnp.int32)
    b2a = b2a.astype(jnp.int32)
    frags_a2b = frags_a2b.astype(jnp.int32)
    frags_b2a = frags_b2a.astype(jnp.int32)
    a2frag = a2frag.astype(jnp.int32)
    counts = jax.ops.segment_sum(jnp.ones((NA,), jnp.float32),
                                 atom_seg, num_segments=N_MOLS)
    fcounts = jax.ops.segment_sum(jnp.ones((FNA,), jnp.float32),
                                  frags_atom_seg, num_segments=N_FRAG_MOLS)

    frags_input, fb = tc_matmul(f_frags_bonds, W_i, both=True)
    ffs = []
    for _ in range(DEPTH - 1):
        fA = sc_gather_sum(fb, frags_a2b, FNAP)
        fAh = tc_matmul(fA, W_h)
        Df = sc_gather_rows(fAh, frags_b2a)
        fb = tc_combine(fb, W_h, frags_input, Df)
        fA2 = sc_gather_sum(fb, frags_a2b, FNAP)
        a_in = jnp.concatenate([f_frags_atoms, fA2[:FNA]], axis=1)
        fh = tc_matmul(a_in, W_o, b=b_o, relu=True)
        ffm = _segmean(fh, frags_atom_seg, N_FRAG_MOLS, FNSP, NSEGP, fcounts)
        ffs.append(jnp.concatenate([jnp.zeros((1, H), jnp.float32), ffm], 0))

    gwbs = []
    for t in range(DEPTH - 1):
        gwtab = tc_matmul(ffs[t], Wf2)
        gat = sc_gather_rows(gwtab, a2frag)
        gwbs.append(sc_gather_rows(gat, b2a))

    inp, message = tc_matmul(f_bonds, W_i, both=True)
    for t in range(DEPTH - 1):
        msg1 = tc_matmul(message, Wf1, b=b_fusion, add=gwbs[t])
        A = sc_gather_sum(msg1, a2b, NAP)
        C = tc_matmul(A, W_h)
        G1 = sc_gather_rows(C, b2a)
        message = tc_combine(msg1, W_h, inp, G1)

    A2 = sc_gather_sum(message, a2b, NAP)
    a_in = jnp.concatenate([f_atoms, A2[:NA]], axis=1)
    atom_hiddens = tc_matmul(a_in, W_o, b=b_o, relu=True)
    mol_vecs = _segmean(atom_hiddens, atom_seg, N_MOLS, NSP, NSEGP, counts)
    return mol_vecs, atom_hiddens, ffs[-1]


def kernel(f_atoms, f_bonds, a2b, b2a, b2revb, atom_seg, f_frags_atoms,
           f_frags_bonds, frags_a2b, frags_b2a, frags_b2revb, frags_atom_seg,
           a2frag, W_i, W_h, W_fusion, b_fusion, W_o, b_o):
    return _run(f_atoms, f_bonds, a2b, b2a, b2revb, atom_seg, f_frags_atoms,
                f_frags_bonds, frags_a2b, frags_b2a, frags_b2revb,
                frags_atom_seg, a2frag, W_i, W_h, W_fusion, b_fusion, W_o, b_o)


# speedup vs baseline: 1.5742x; 1.5742x over previous
import functools
import jax
import jax.numpy as jnp
from jax import lax
from jax.experimental import pallas as pl
from jax.experimental.pallas import tpu as pltpu
from jax.experimental.pallas import tpu_sc as plsc

DEPTH = 3
N_MOLS = 500
N_FRAG_MOLS = 500
BN = 512
NC, NS = 2, 16
NW = NC * NS


def _cdivmul(n, m):
    return -(-n // m) * m


def tc_matmul(x, w, b=None, add=None, addcat=None, relu=False, both=False):
    n, k = x.shape
    h = w.shape[1]
    npad = -n % BN
    if npad:
        x = jnp.pad(x, ((0, npad), (0, 0)))
        if add is not None:
            add = jnp.pad(add, ((0, npad), (0, 0)))
    np_ = x.shape[0]
    grid = (np_ // BN,)
    xspec = pl.BlockSpec((BN, k), lambda i: (i, 0))
    wspec = pl.BlockSpec((k, h), lambda i: (0, 0))
    bspec = pl.BlockSpec((1, h), lambda i: (0, 0))
    ospec = pl.BlockSpec((BN, h), lambda i: (i, 0))
    in_specs = [xspec, wspec]
    args = [x, w]
    if b is not None:
        in_specs.append(bspec)
        args.append(b.reshape(1, h))
    if add is not None:
        in_specs.append(ospec)
        args.append(add)
    if addcat is not None:
        acat, tcol = addcat
        in_specs.append(pl.BlockSpec((BN, h), lambda i, _t=tcol: (i, _t)))
        args.append(acat)
    if both:
        out_shape = (jax.ShapeDtypeStruct((np_, h), jnp.float32),) * 2
        out_specs = (ospec, ospec)
    else:
        out_shape = jax.ShapeDtypeStruct((np_, h), jnp.float32)
        out_specs = ospec

    def body(*refs):
        it = iter(refs)
        x_ref = next(it)
        w_ref = next(it)
        b_ref = next(it) if b is not None else None
        a_ref = next(it) if add is not None else None
        ac_ref = next(it) if addcat is not None else None
        acc = jnp.dot(x_ref[...], w_ref[...], preferred_element_type=jnp.float32)
        if b_ref is not None:
            acc = acc + b_ref[...]
        if a_ref is not None:
            acc = acc + a_ref[...]
        if ac_ref is not None:
            acc = acc + ac_ref[...]
        if both:
            next(it)[...] = acc
            next(it)[...] = jnp.maximum(acc, 0.0)
        elif relu:
            next(it)[...] = jnp.maximum(acc, 0.0)
        else:
            next(it)[...] = acc

    out = pl.pallas_call(
        body, grid=grid, in_specs=in_specs, out_specs=out_specs,
        out_shape=out_shape,
        compiler_params=pltpu.CompilerParams(
            dimension_semantics=("parallel",)))(*args)
    if both:
        return (out[0][:n], out[1][:n]) if npad else out
    return out[:n] if npad else out


def _pairswap(m):
    up = jnp.concatenate([m[1:], m[:1]], axis=0)
    dn = jnp.concatenate([m[-1:], m[:-1]], axis=0)
    rows = lax.broadcasted_iota(jnp.int32, m.shape, 0)
    return jnp.where(rows % 2 == 0, up, dn)


def tc_combine(x, w, inp, g1):
    n, h = x.shape
    assert n % BN == 0
    grid = (n // BN,)
    spec = pl.BlockSpec((BN, h), lambda i: (i, 0))
    wspec = pl.BlockSpec((h, h), lambda i: (0, 0))

    def body(x_ref, w_ref, inp_ref, g1_ref, o_ref):
        m = jnp.dot(x_ref[...], w_ref[...], preferred_element_type=jnp.float32)
        o_ref[...] = jnp.maximum(inp_ref[...] + g1_ref[...] - _pairswap(m), 0.0)

    return pl.pallas_call(
        body, grid=grid, in_specs=[spec, wspec, spec, spec], out_specs=spec,
        out_shape=jax.ShapeDtypeStruct((n, h), jnp.float32),
        compiler_params=pltpu.CompilerParams(
            dimension_semantics=("parallel",)))(x, w, inp, g1)


def _sc_mesh():
    return plsc.VectorSubcoreMesh(core_axis_name="c", subcore_axis_name="s")


def _wid():
    return lax.axis_index("s") * NC + lax.axis_index("c")


def sc_gather_rows(table, idx, replicate=False):
    v, hc = table.shape
    s = 2 if hc <= 128 else 1
    ch = s * 128
    sc_rows = 1024
    rounds = sc_rows // ch
    b = idx.shape[0]
    bp = _cdivmul(b, NW * sc_rows)
    if bp != b:
        pad = jnp.arange(bp - b, dtype=jnp.int32) % v
        idx = jnp.concatenate([idx, pad])
    if replicate:
        table = jnp.tile(table, (NW, 1))
    idx2 = idx.reshape(bp // 128, 128)
    bpw = bp // NW
    nch = bpw // sc_rows

    @functools.partial(
        pl.kernel,
        out_type=jax.ShapeDtypeStruct((bp, hc), jnp.float32),
        mesh=_sc_mesh(),
        scratch_types=[pltpu.VMEM((8, 128), jnp.int32),
                       pltpu.VMEM((ch, hc), jnp.float32),
                       pltpu.VMEM((ch, hc), jnp.float32),
                       pltpu.SemaphoreType.DMA,
                       pltpu.SemaphoreType.DMA],
    )
    def k(table_h, idx_h, out_h, idx_v, rows_a, rows_b, isem, gsem):
        base = _wid() * bpw

        def idx_cp(i):
            off = pl.multiple_of(base + i * sc_rows, 1024)
            return pltpu.make_async_copy(
                idx_h.at[pl.ds(pl.multiple_of(off // 128, 8), 8)], idx_v, isem)

        def fire(buf, rr):
            return [pltpu.async_copy(table_h.at[idx_v.at[rr * s + j]],
                                     buf.at[pl.ds(j * 128, 128)], gsem)
                    for j in range(s)]

        idx_cp(0).start()

        def chunk(i, c):
            off = pl.multiple_of(base + i * sc_rows, 1024)
            idx_cp(i).wait()
            if replicate:
                woff = _wid() * v
                for r8 in range(8):
                    for kk in range(8):
                        sl = pl.ds(kk * 16, 16)
                        idx_v[r8, sl] = idx_v[r8, sl] + woff
            cps = fire(rows_a, 0)
            for rr in range(rounds):
                buf = rows_a if rr % 2 == 0 else rows_b
                for cp in cps:
                    cp.wait()
                if rr + 1 < rounds:
                    cps = fire(rows_b if rr % 2 == 0 else rows_a, rr + 1)
                pltpu.sync_copy(
                    buf,
                    out_h.at[pl.ds(pl.multiple_of(off + rr * ch, ch), ch)])

            @pl.when(i + 1 < nch)
            def _():
                idx_cp(i + 1).start()

            return c

        lax.fori_loop(0, nch, chunk, 0)

    return k(table, idx2)


def sc_gather_sum(table, idx2d, nap):
    na, nb = idx2d.shape
    v, h = table.shape
    s = 2 if h <= 128 else 1
    ch = s * 128
    arh = ch // nb
    asc = 1024 // nb
    rounds = 1024 // ch
    assert nap % (NW * asc) == 0
    idx = idx2d
    if nap != na:
        idx = jnp.pad(idx, ((0, nap - na), (0, 0)))
    idxf = idx.reshape(nap * nb // 128, 128)
    apw = nap // NW
    nch = apw // asc
    hb = h // 16

    @functools.partial(
        pl.kernel,
        out_type=jax.ShapeDtypeStruct((nap, h), jnp.float32),
        mesh=_sc_mesh(),
        scratch_types=[pltpu.VMEM((8, 128), jnp.int32),
                       pltpu.VMEM((ch, h), jnp.float32),
                       pltpu.VMEM((ch, h), jnp.float32),
                       pltpu.VMEM((asc, h), jnp.float32),
                       pltpu.SemaphoreType.DMA,
                       pltpu.SemaphoreType.DMA],
    )
    def k(table_h, idx_h, out_h, idx_v, rows_a, rows_b, out_v, isem, gsem):
        base = _wid() * apw

        def idx_cp(i):
            aoff = pl.multiple_of(base + i * asc, asc)
            return pltpu.make_async_copy(
                idx_h.at[pl.ds(pl.multiple_of(aoff * nb // 128, 8), 8)],
                idx_v, isem)

        def fire(buf, rr):
            for j in range(s):
                pltpu.make_async_copy(table_h.at[idx_v.at[rr * s + j]],
                                      buf.at[pl.ds(j * 128, 128)],
                                      gsem).start()

        def drain(buf):
            for j in range(s):
                pltpu.make_async_copy(table_h.at[idx_v.at[0]],
                                      buf.at[pl.ds(j * 128, 128)],
                                      gsem).wait()

        def consume(buf, rr):
            def atom(a, c2):
                r0 = a * nb
                for hh in range(hb):
                    sl = pl.ds(hh * 16, 16)
                    acc = buf[r0, sl]
                    for j in range(1, nb):
                        acc = acc + buf[r0 + j, sl]
                    out_v[rr * arh + a, sl] = acc
                return c2

            lax.fori_loop(0, arh, atom, 0)

        idx_cp(0).start()

        def chunk(i, c):
            aoff = pl.multiple_of(base + i * asc, asc)
            idx_cp(i).wait()
            fire(rows_a, 0)

            def rpair(rp, c2):
                rr0 = 2 * rp
                drain(rows_a)
                fire(rows_b, rr0 + 1)
                consume(rows_a, rr0)
                drain(rows_b)

                @pl.when(rr0 + 2 < rounds)
                def _():
                    fire(rows_a, rr0 + 2)

                consume(rows_b, rr0 + 1)
                return c2

            lax.fori_loop(0, rounds // 2, rpair, 0)
            pltpu.sync_copy(out_v,
                            out_h.at[pl.ds(pl.multiple_of(aoff, asc), asc)])

            @pl.when(i + 1 < nch)
            def _():
                idx_cp(i + 1).start()

            return c

        lax.fori_loop(0, nch, chunk, 0)

    return k(table, idxf)


def sc_segsum(x, seg, nsegp):
    npts, h = x.shape
    assert npts % (NW * 128) == 0
    apw = npts // NW
    nch = apw // 128
    zeros = jnp.zeros((nsegp, h), jnp.float32)

    @functools.partial(
        pl.kernel,
        out_type=jax.ShapeDtypeStruct((NC, nsegp, h), jnp.float32),
        mesh=_sc_mesh(),
        scratch_types=[pltpu.VMEM((128,), jnp.int32),
                       pltpu.VMEM((128, h), jnp.float32),
                       pltpu.VMEM_SHARED((nsegp, h), jnp.float32)],
    )
    def k(x_h, seg_h, z_h, out_h, seg_v, x_v, acc_sh):
        sid = lax.axis_index("s")
        cid = lax.axis_index("c")
        base = _wid() * apw

        @pl.when(sid == 0)
        def _():
            pltpu.sync_copy(z_h, acc_sh)

        plsc.subcore_barrier()

        def chunk(i, c):
            off = pl.multiple_of(base + i * 128, 128)
            pltpu.sync_copy(seg_h.at[pl.ds(off, 128)], seg_v)
            pltpu.sync_copy(x_h.at[pl.ds(off, 128)], x_v)
            pltpu.sync_copy(x_v, acc_sh.at[seg_v], add=True)
            return c

        lax.fori_loop(0, nch, chunk, 0)
        plsc.subcore_barrier()

        @pl.when(sid == 0)
        def _():
            pltpu.sync_copy(acc_sh, out_h.at[cid])

    return k(x, seg, zeros)


def _segmean(x, seg, n, npad, nsegp, counts):
    xp = jnp.pad(x, ((0, npad - x.shape[0]), (0, 0)))
    sp = jnp.pad(seg.astype(jnp.int32), (0, npad - seg.shape[0]))
    parts = sc_segsum(xp, sp, nsegp)
    sums = parts[0, :n] + parts[1, :n]
    return jnp.where(counts[:, None] > 0,
                     sums / jnp.maximum(counts, 1.0)[:, None], 0.0)


@jax.jit
def _run(f_atoms, f_bonds, a2b, b2a, b2revb, atom_seg,
         f_frags_atoms, f_frags_bonds, frags_a2b, frags_b2a, frags_b2revb,
         frags_atom_seg, a2frag, W_i, W_h, W_fusion, b_fusion, W_o, b_o):
    H = W_h.shape[0]
    NA, MAXNB = a2b.shape
    NB_ = b2a.shape[0]
    FNA, FMAXNB = frags_a2b.shape
    NAP = _cdivmul(NA, 1024)
    FNAP = _cdivmul(FNA, 2048)
    NSP = _cdivmul(NA, NW * 128)
    FNSP = _cdivmul(FNA, NW * 128)
    NSEGP = _cdivmul(N_MOLS, 8)

    Wf1, Wf2 = W_fusion[:H], W_fusion[H:]

    a2b = a2b.astype(jnp.int32)
    b2a = b2a.astype(jnp.int32)
    frags_a2b = frags_a2b.astype(jnp.int32)
    frags_b2a = frags_b2a.astype(jnp.int32)
    a2frag = a2frag.astype(jnp.int32)
    counts = jax.ops.segment_sum(jnp.ones((NA,), jnp.float32),
                                 atom_seg, num_segments=N_MOLS)
    fcounts = jax.ops.segment_sum(jnp.ones((FNA,), jnp.float32),
                                  frags_atom_seg, num_segments=N_FRAG_MOLS)

    frags_input, fb = tc_matmul(f_frags_bonds, W_i, both=True)
    ffs = []
    for _ in range(DEPTH - 1):
        fA = sc_gather_sum(fb, frags_a2b, FNAP)
        fAh = tc_matmul(fA, W_h)
        Df = sc_gather_rows(fAh, frags_b2a)
        fb = tc_combine(fb, W_h, frags_input, Df)
        fA2 = sc_gather_sum(fb, frags_a2b, FNAP)
        a_in = jnp.concatenate([f_frags_atoms, fA2[:FNA]], axis=1)
        fh = tc_matmul(a_in, W_o, b=b_o, relu=True)
        ffm = _segmean(fh, frags_atom_seg, N_FRAG_MOLS, FNSP, NSEGP, fcounts)
        ffs.append(jnp.concatenate([jnp.zeros((1, H), jnp.float32), ffm], 0))

    gwbs = []
    for t in range(DEPTH - 1):
        gwtab = tc_matmul(ffs[t], Wf2)
        gat = sc_gather_rows(gwtab, a2frag, replicate=True)
        gwbs.append(sc_gather_rows(gat, b2a))

    inp, message = tc_matmul(f_bonds, W_i, both=True)
    for t in range(DEPTH - 1):
        msg1 = tc_matmul(message, Wf1, b=b_fusion, add=gwbs[t])
        A = sc_gather_sum(msg1, a2b, NAP)
        C = tc_matmul(A, W_h)
        G1 = sc_gather_rows(C, b2a)
        message = tc_combine(msg1, W_h, inp, G1)

    A2 = sc_gather_sum(message, a2b, NAP)
    a_in = jnp.concatenate([f_atoms, A2[:NA]], axis=1)
    atom_hiddens = tc_matmul(a_in, W_o, b=b_o, relu=True)
    mol_vecs = _segmean(atom_hiddens, atom_seg, N_MOLS, NSP, NSEGP, counts)
    return mol_vecs, atom_hiddens, ffs[-1]


def kernel(f_atoms, f_bonds, a2b, b2a, b2revb, atom_seg, f_frags_atoms,
           f_frags_bonds, frags_a2b, frags_b2a, frags_b2revb, frags_atom_seg,
           a2frag, W_i, W_h, W_fusion, b_fusion, W_o, b_o):
    return _run(f_atoms, f_bonds, a2b, b2a, b2revb, atom_seg, f_frags_atoms,
                f_frags_bonds, frags_a2b, frags_b2a, frags_b2revb,
                frags_atom_seg, a2frag, W_i, W_h, W_fusion, b_fusion, W_o, b_o)


# speedup vs baseline: 1.7660x; 1.1219x over previous
import functools
import jax
import jax.numpy as jnp
from jax import lax
from jax.experimental import pallas as pl
from jax.experimental.pallas import tpu as pltpu
from jax.experimental.pallas import tpu_sc as plsc

DEPTH = 3
N_MOLS = 500
N_FRAG_MOLS = 500
BN = 512
NC, NS = 2, 16
NW = NC * NS


def _cdivmul(n, m):
    return -(-n // m) * m


def tc_matmul(x, w, b=None, add=None, addcat=None, relu=False, both=False):
    n, k = x.shape
    h = w.shape[1]
    npad = -n % BN
    if npad:
        x = jnp.pad(x, ((0, npad), (0, 0)))
        if add is not None:
            add = jnp.pad(add, ((0, npad), (0, 0)))
    np_ = x.shape[0]
    grid = (np_ // BN,)
    xspec = pl.BlockSpec((BN, k), lambda i: (i, 0))
    wspec = pl.BlockSpec((k, h), lambda i: (0, 0))
    bspec = pl.BlockSpec((1, h), lambda i: (0, 0))
    ospec = pl.BlockSpec((BN, h), lambda i: (i, 0))
    in_specs = [xspec, wspec]
    args = [x, w]
    if b is not None:
        in_specs.append(bspec)
        args.append(b.reshape(1, h))
    if add is not None:
        in_specs.append(ospec)
        args.append(add)
    if addcat is not None:
        acat, tcol = addcat
        in_specs.append(pl.BlockSpec((BN, h), lambda i, _t=tcol: (i, _t)))
        args.append(acat)
    if both:
        out_shape = (jax.ShapeDtypeStruct((np_, h), jnp.float32),) * 2
        out_specs = (ospec, ospec)
    else:
        out_shape = jax.ShapeDtypeStruct((np_, h), jnp.float32)
        out_specs = ospec

    def body(*refs):
        it = iter(refs)
        x_ref = next(it)
        w_ref = next(it)
        b_ref = next(it) if b is not None else None
        a_ref = next(it) if add is not None else None
        ac_ref = next(it) if addcat is not None else None
        acc = jnp.dot(x_ref[...], w_ref[...], preferred_element_type=jnp.float32)
        if b_ref is not None:
            acc = acc + b_ref[...]
        if a_ref is not None:
            acc = acc + a_ref[...]
        if ac_ref is not None:
            acc = acc + ac_ref[...]
        if both:
            next(it)[...] = acc
            next(it)[...] = jnp.maximum(acc, 0.0)
        elif relu:
            next(it)[...] = jnp.maximum(acc, 0.0)
        else:
            next(it)[...] = acc

    out = pl.pallas_call(
        body, grid=grid, in_specs=in_specs, out_specs=out_specs,
        out_shape=out_shape,
        compiler_params=pltpu.CompilerParams(
            dimension_semantics=("parallel",)))(*args)
    if both:
        return (out[0][:n], out[1][:n]) if npad else out
    return out[:n] if npad else out


def _pairswap(m):
    up = jnp.concatenate([m[1:], m[:1]], axis=0)
    dn = jnp.concatenate([m[-1:], m[:-1]], axis=0)
    rows = lax.broadcasted_iota(jnp.int32, m.shape, 0)
    return jnp.where(rows % 2 == 0, up, dn)


def tc_init(f_bonds, W_i, Wf1, gwb0, b_fusion):
    n, k = f_bonds.shape
    h = W_i.shape[1]
    assert n % BN == 0
    grid = (n // BN,)
    xspec = pl.BlockSpec((BN, k), lambda i: (i, 0))
    wspec = pl.BlockSpec((k, h), lambda i: (0, 0))
    w2spec = pl.BlockSpec((h, h), lambda i: (0, 0))
    spec = pl.BlockSpec((BN, h), lambda i: (i, 0))
    bspec = pl.BlockSpec((1, h), lambda i: (0, 0))

    def body(x_ref, wi_ref, wf_ref, g_ref, b_ref, inp_ref, m1_ref):
        acc = jnp.dot(x_ref[...], wi_ref[...],
                      preferred_element_type=jnp.float32)
        inp_ref[...] = acc
        m0 = jnp.maximum(acc, 0.0)
        m1_ref[...] = (jnp.dot(m0, wf_ref[...],
                               preferred_element_type=jnp.float32)
                       + g_ref[...] + b_ref[...])

    return pl.pallas_call(
        body, grid=grid,
        in_specs=[xspec, wspec, w2spec, spec, bspec],
        out_specs=(spec, spec),
        out_shape=(jax.ShapeDtypeStruct((n, h), jnp.float32),) * 2,
        compiler_params=pltpu.CompilerParams(
            dimension_semantics=("parallel",)))(
            f_bonds, W_i, Wf1, gwb0, b_fusion.reshape(1, h))


def tc_step(msg1, W_h, Wf1, inp, g1, gwb_next, b_fusion):
    n, h = msg1.shape
    assert n % BN == 0
    grid = (n // BN,)
    spec = pl.BlockSpec((BN, h), lambda i: (i, 0))
    wspec = pl.BlockSpec((h, h), lambda i: (0, 0))
    bspec = pl.BlockSpec((1, h), lambda i: (0, 0))

    def body(x_ref, wh_ref, wf_ref, inp_ref, g1_ref, g_ref, b_ref, o_ref):
        m = jnp.dot(x_ref[...], wh_ref[...],
                    preferred_element_type=jnp.float32)
        nxt = jnp.maximum(inp_ref[...] + g1_ref[...] - _pairswap(m), 0.0)
        o_ref[...] = (jnp.dot(nxt, wf_ref[...],
                              preferred_element_type=jnp.float32)
                      + g_ref[...] + b_ref[...])

    return pl.pallas_call(
        body, grid=grid,
        in_specs=[spec, wspec, wspec, spec, spec, spec, bspec],
        out_specs=spec,
        out_shape=jax.ShapeDtypeStruct((n, h), jnp.float32),
        compiler_params=pltpu.CompilerParams(
            dimension_semantics=("parallel",)))(
            msg1, W_h, Wf1, inp, g1, gwb_next, b_fusion.reshape(1, h))


def tc_combine(x, w, inp, g1):
    n, h = x.shape
    assert n % BN == 0
    grid = (n // BN,)
    spec = pl.BlockSpec((BN, h), lambda i: (i, 0))
    wspec = pl.BlockSpec((h, h), lambda i: (0, 0))

    def body(x_ref, w_ref, inp_ref, g1_ref, o_ref):
        m = jnp.dot(x_ref[...], w_ref[...], preferred_element_type=jnp.float32)
        o_ref[...] = jnp.maximum(inp_ref[...] + g1_ref[...] - _pairswap(m), 0.0)

    return pl.pallas_call(
        body, grid=grid, in_specs=[spec, wspec, spec, spec], out_specs=spec,
        out_shape=jax.ShapeDtypeStruct((n, h), jnp.float32),
        compiler_params=pltpu.CompilerParams(
            dimension_semantics=("parallel",)))(x, w, inp, g1)


def _sc_mesh():
    return plsc.VectorSubcoreMesh(core_axis_name="c", subcore_axis_name="s")


def _wid():
    return lax.axis_index("s") * NC + lax.axis_index("c")


def sc_gather_rows(table, idx, replicate=False):
    v, hc = table.shape
    s = 2 if hc <= 128 else 1
    ch = s * 128
    sc_rows = 1024
    rounds = sc_rows // ch
    b = idx.shape[0]
    bp = _cdivmul(b, NW * sc_rows)
    if bp != b:
        pad = jnp.arange(bp - b, dtype=jnp.int32) % v
        idx = jnp.concatenate([idx, pad])
    if replicate:
        table = jnp.tile(table, (NW, 1))
    idx2 = idx.reshape(bp // 128, 128)
    bpw = bp // NW
    nch = bpw // sc_rows

    @functools.partial(
        pl.kernel,
        out_type=jax.ShapeDtypeStruct((bp, hc), jnp.float32),
        mesh=_sc_mesh(),
        scratch_types=[pltpu.VMEM((8, 128), jnp.int32),
                       pltpu.VMEM((ch, hc), jnp.float32),
                       pltpu.VMEM((ch, hc), jnp.float32),
                       pltpu.SemaphoreType.DMA,
                       pltpu.SemaphoreType.DMA],
    )
    def k(table_h, idx_h, out_h, idx_v, rows_a, rows_b, isem, gsem):
        base = _wid() * bpw

        def idx_cp(i):
            off = pl.multiple_of(base + i * sc_rows, 1024)
            return pltpu.make_async_copy(
                idx_h.at[pl.ds(pl.multiple_of(off // 128, 8), 8)], idx_v, isem)

        def fire(buf, rr):
            return [pltpu.async_copy(table_h.at[idx_v.at[rr * s + j]],
                                     buf.at[pl.ds(j * 128, 128)], gsem)
                    for j in range(s)]

        idx_cp(0).start()

        def chunk(i, c):
            off = pl.multiple_of(base + i * sc_rows, 1024)
            idx_cp(i).wait()
            if replicate:
                woff = _wid() * v
                for r8 in range(8):
                    for kk in range(8):
                        sl = pl.ds(kk * 16, 16)
                        idx_v[r8, sl] = idx_v[r8, sl] + woff
            cps = fire(rows_a, 0)
            for rr in range(rounds):
                buf = rows_a if rr % 2 == 0 else rows_b
                for cp in cps:
                    cp.wait()
                if rr + 1 < rounds:
                    cps = fire(rows_b if rr % 2 == 0 else rows_a, rr + 1)
                pltpu.sync_copy(
                    buf,
                    out_h.at[pl.ds(pl.multiple_of(off + rr * ch, ch), ch)])

            @pl.when(i + 1 < nch)
            def _():
                idx_cp(i + 1).start()

            return c

        lax.fori_loop(0, nch, chunk, 0)

    return k(table, idx2)


def sc_gather_sum(table, idx2d, nap):
    na, nb = idx2d.shape
    v, h = table.shape
    s = 2 if h <= 128 else 1
    ch = s * 128
    arh = ch // nb
    asc = 1024 // nb
    rounds = 1024 // ch
    assert nap % (NW * asc) == 0
    idx = idx2d
    if nap != na:
        idx = jnp.pad(idx, ((0, nap - na), (0, 0)))
    idxf = idx.reshape(nap * nb // 128, 128)
    apw = nap // NW
    nch = apw // asc
    hb = h // 16

    @functools.partial(
        pl.kernel,
        out_type=jax.ShapeDtypeStruct((nap, h), jnp.float32),
        mesh=_sc_mesh(),
        scratch_types=[pltpu.VMEM((8, 128), jnp.int32),
                       pltpu.VMEM((ch, h), jnp.float32),
                       pltpu.VMEM((ch, h), jnp.float32),
                       pltpu.VMEM((asc, h), jnp.float32),
                       pltpu.SemaphoreType.DMA,
                       pltpu.SemaphoreType.DMA],
    )
    def k(table_h, idx_h, out_h, idx_v, rows_a, rows_b, out_v, isem, gsem):
        base = _wid() * apw

        def idx_cp(i):
            aoff = pl.multiple_of(base + i * asc, asc)
            return pltpu.make_async_copy(
                idx_h.at[pl.ds(pl.multiple_of(aoff * nb // 128, 8), 8)],
                idx_v, isem)

        def fire(buf, rr):
            for j in range(s):
                pltpu.make_async_copy(table_h.at[idx_v.at[rr * s + j]],
                                      buf.at[pl.ds(j * 128, 128)],
                                      gsem).start()

        def drain(buf):
            for j in range(s):
                pltpu.make_async_copy(table_h.at[idx_v.at[0]],
                                      buf.at[pl.ds(j * 128, 128)],
                                      gsem).wait()

        def consume(buf, rr):
            def atom(a, c2):
                r0 = a * nb
                for hh in range(hb):
                    sl = pl.ds(hh * 16, 16)
                    acc = buf[r0, sl]
                    for j in range(1, nb):
                        acc = acc + buf[r0 + j, sl]
                    out_v[rr * arh + a, sl] = acc
                return c2

            lax.fori_loop(0, arh, atom, 0)

        idx_cp(0).start()

        def chunk(i, c):
            aoff = pl.multiple_of(base + i * asc, asc)
            idx_cp(i).wait()
            fire(rows_a, 0)

            def rpair(rp, c2):
                rr0 = 2 * rp
                drain(rows_a)
                fire(rows_b, rr0 + 1)
                consume(rows_a, rr0)
                drain(rows_b)

                @pl.when(rr0 + 2 < rounds)
                def _():
                    fire(rows_a, rr0 + 2)

                consume(rows_b, rr0 + 1)
                return c2

            lax.fori_loop(0, rounds // 2, rpair, 0)
            pltpu.sync_copy(out_v,
                            out_h.at[pl.ds(pl.multiple_of(aoff, asc), asc)])

            @pl.when(i + 1 < nch)
            def _():
                idx_cp(i + 1).start()

            return c

        lax.fori_loop(0, nch, chunk, 0)

    return k(table, idxf)


def sc_segsum(x, seg, nsegp):
    npts, h = x.shape
    assert npts % (NW * 128) == 0
    apw = npts // NW
    nch = apw // 128
    zeros = jnp.zeros((nsegp, h), jnp.float32)

    @functools.partial(
        pl.kernel,
        out_type=jax.ShapeDtypeStruct((NC, nsegp, h), jnp.float32),
        mesh=_sc_mesh(),
        scratch_types=[pltpu.VMEM((128,), jnp.int32),
                       pltpu.VMEM((128, h), jnp.float32),
                       pltpu.VMEM_SHARED((nsegp, h), jnp.float32)],
    )
    def k(x_h, seg_h, z_h, out_h, seg_v, x_v, acc_sh):
        sid = lax.axis_index("s")
        cid = lax.axis_index("c")
        base = _wid() * apw

        @pl.when(sid == 0)
        def _():
            pltpu.sync_copy(z_h, acc_sh)

        plsc.subcore_barrier()

        def chunk(i, c):
            off = pl.multiple_of(base + i * 128, 128)
            pltpu.sync_copy(seg_h.at[pl.ds(off, 128)], seg_v)
            pltpu.sync_copy(x_h.at[pl.ds(off, 128)], x_v)
            pltpu.sync_copy(x_v, acc_sh.at[seg_v], add=True)
            return c

        lax.fori_loop(0, nch, chunk, 0)
        plsc.subcore_barrier()

        @pl.when(sid == 0)
        def _():
            pltpu.sync_copy(acc_sh, out_h.at[cid])

    return k(x, seg, zeros)


def _segmean(x, seg, n, npad, nsegp, counts):
    xp = jnp.pad(x, ((0, npad - x.shape[0]), (0, 0)))
    sp = jnp.pad(seg.astype(jnp.int32), (0, npad - seg.shape[0]))
    parts = sc_segsum(xp, sp, nsegp)
    sums = parts[0, :n] + parts[1, :n]
    return jnp.where(counts[:, None] > 0,
                     sums / jnp.maximum(counts, 1.0)[:, None], 0.0)


@jax.jit
def _run(f_atoms, f_bonds, a2b, b2a, b2revb, atom_seg,
         f_frags_atoms, f_frags_bonds, frags_a2b, frags_b2a, frags_b2revb,
         frags_atom_seg, a2frag, W_i, W_h, W_fusion, b_fusion, W_o, b_o):
    H = W_h.shape[0]
    NA, MAXNB = a2b.shape
    NB_ = b2a.shape[0]
    FNA, FMAXNB = frags_a2b.shape
    NAP = _cdivmul(NA, 1024)
    FNAP = _cdivmul(FNA, 2048)
    NSP = _cdivmul(NA, NW * 128)
    FNSP = _cdivmul(FNA, NW * 128)
    NSEGP = _cdivmul(N_MOLS, 8)

    Wf1, Wf2 = W_fusion[:H], W_fusion[H:]

    a2b = a2b.astype(jnp.int32)
    b2a = b2a.astype(jnp.int32)
    frags_a2b = frags_a2b.astype(jnp.int32)
    frags_b2a = frags_b2a.astype(jnp.int32)
    a2frag = a2frag.astype(jnp.int32)
    counts = jax.ops.segment_sum(jnp.ones((NA,), jnp.float32),
                                 atom_seg, num_segments=N_MOLS)
    fcounts = jax.ops.segment_sum(jnp.ones((FNA,), jnp.float32),
                                  frags_atom_seg, num_segments=N_FRAG_MOLS)

    frags_input, fb = tc_matmul(f_frags_bonds, W_i, both=True)
    ffs = []
    for _ in range(DEPTH - 1):
        fA = sc_gather_sum(fb, frags_a2b, FNAP)
        fAh = tc_matmul(fA, W_h)
        Df = sc_gather_rows(fAh, frags_b2a)
        fb = tc_combine(fb, W_h, frags_input, Df)
        fA2 = sc_gather_sum(fb, frags_a2b, FNAP)
        a_in = jnp.concatenate([f_frags_atoms, fA2[:FNA]], axis=1)
        fh = tc_matmul(a_in, W_o, b=b_o, relu=True)
        ffm = _segmean(fh, frags_atom_seg, N_FRAG_MOLS, FNSP, NSEGP, fcounts)
        ffs.append(jnp.concatenate([jnp.zeros((1, H), jnp.float32), ffm], 0))

    gwbs = []
    for t in range(DEPTH - 1):
        gwtab = tc_matmul(ffs[t], Wf2)
        gat = sc_gather_rows(gwtab, a2frag, replicate=True)
        gwbs.append(sc_gather_rows(gat, b2a))

    inp, msg1 = tc_init(f_bonds, W_i, Wf1, gwbs[0], b_fusion)
    for t in range(DEPTH - 1):
        A = sc_gather_sum(msg1, a2b, NAP)
        C = tc_matmul(A, W_h)
        G1 = sc_gather_rows(C, b2a)
        if t + 1 < DEPTH - 1:
            msg1 = tc_step(msg1, W_h, Wf1, inp, G1, gwbs[t + 1], b_fusion)
        else:
            message = tc_combine(msg1, W_h, inp, G1)

    A2 = sc_gather_sum(message, a2b, NAP)
    a_in = jnp.concatenate([f_atoms, A2[:NA]], axis=1)
    atom_hiddens = tc_matmul(a_in, W_o, b=b_o, relu=True)
    mol_vecs = _segmean(atom_hiddens, atom_seg, N_MOLS, NSP, NSEGP, counts)
    return mol_vecs, atom_hiddens, ffs[-1]


def kernel(f_atoms, f_bonds, a2b, b2a, b2revb, atom_seg, f_frags_atoms,
           f_frags_bonds, frags_a2b, frags_b2a, frags_b2revb, frags_atom_seg,
           a2frag, W_i, W_h, W_fusion, b_fusion, W_o, b_o):
    return _run(f_atoms, f_bonds, a2b, b2a, b2revb, atom_seg, f_frags_atoms,
                f_frags_bonds, frags_a2b, frags_b2a, frags_b2revb,
                frags_atom_seg, a2frag, W_i, W_h, W_fusion, b_fusion, W_o, b_o)


# speedup vs baseline: 2.1176x; 1.1991x over previous
import functools
import jax
import jax.numpy as jnp
from jax import lax
from jax.experimental import pallas as pl
from jax.experimental.pallas import tpu as pltpu
from jax.experimental.pallas import tpu_sc as plsc

DEPTH = 3
N_MOLS = 500
N_FRAG_MOLS = 500
BN = 1280
NC, NS = 2, 16
NW = NC * NS


def _cdivmul(n, m):
    return -(-n // m) * m


def tc_matmul(x, w, b=None, add=None, addcat=None, relu=False, both=False):
    n, k = x.shape
    h = w.shape[1]
    npad = -n % BN
    if npad:
        x = jnp.pad(x, ((0, npad), (0, 0)))
        if add is not None:
            add = jnp.pad(add, ((0, npad), (0, 0)))
    np_ = x.shape[0]
    grid = (np_ // BN,)
    xspec = pl.BlockSpec((BN, k), lambda i: (i, 0))
    wspec = pl.BlockSpec((k, h), lambda i: (0, 0))
    bspec = pl.BlockSpec((1, h), lambda i: (0, 0))
    ospec = pl.BlockSpec((BN, h), lambda i: (i, 0))
    in_specs = [xspec, wspec]
    args = [x, w]
    if b is not None:
        in_specs.append(bspec)
        args.append(b.reshape(1, h))
    if add is not None:
        in_specs.append(ospec)
        args.append(add)
    if addcat is not None:
        acat, tcol = addcat
        in_specs.append(pl.BlockSpec((BN, h), lambda i, _t=tcol: (i, _t)))
        args.append(acat)
    if both:
        out_shape = (jax.ShapeDtypeStruct((np_, h), jnp.float32),) * 2
        out_specs = (ospec, ospec)
    else:
        out_shape = jax.ShapeDtypeStruct((np_, h), jnp.float32)
        out_specs = ospec

    def body(*refs):
        it = iter(refs)
        x_ref = next(it)
        w_ref = next(it)
        b_ref = next(it) if b is not None else None
        a_ref = next(it) if add is not None else None
        ac_ref = next(it) if addcat is not None else None
        acc = jnp.dot(x_ref[...], w_ref[...], preferred_element_type=jnp.float32)
        if b_ref is not None:
            acc = acc + b_ref[...]
        if a_ref is not None:
            acc = acc + a_ref[...]
        if ac_ref is not None:
            acc = acc + ac_ref[...]
        if both:
            next(it)[...] = acc
            next(it)[...] = jnp.maximum(acc, 0.0)
        elif relu:
            next(it)[...] = jnp.maximum(acc, 0.0)
        else:
            next(it)[...] = acc

    out = pl.pallas_call(
        body, grid=grid, in_specs=in_specs, out_specs=out_specs,
        out_shape=out_shape,
        compiler_params=pltpu.CompilerParams(
            dimension_semantics=("parallel",)))(*args)
    if both:
        return (out[0][:n], out[1][:n]) if npad else out
    return out[:n] if npad else out


def _pairswap(m):
    up = jnp.concatenate([m[1:], m[:1]], axis=0)
    dn = jnp.concatenate([m[-1:], m[:-1]], axis=0)
    rows = lax.broadcasted_iota(jnp.int32, m.shape, 0)
    return jnp.where(rows % 2 == 0, up, dn)


def tc_init(f_bonds, W_i, Wf1, gwb0, b_fusion):
    n, k = f_bonds.shape
    h = W_i.shape[1]
    assert n % BN == 0
    grid = (n // BN,)
    xspec = pl.BlockSpec((BN, k), lambda i: (i, 0))
    wspec = pl.BlockSpec((k, h), lambda i: (0, 0))
    w2spec = pl.BlockSpec((h, h), lambda i: (0, 0))
    spec = pl.BlockSpec((BN, h), lambda i: (i, 0))
    bspec = pl.BlockSpec((1, h), lambda i: (0, 0))

    def body(x_ref, wi_ref, wf_ref, g_ref, b_ref, inp_ref, m1_ref):
        acc = jnp.dot(x_ref[...], wi_ref[...],
                      preferred_element_type=jnp.float32)
        inp_ref[...] = acc
        m0 = jnp.maximum(acc, 0.0)
        m1_ref[...] = (jnp.dot(m0, wf_ref[...],
                               preferred_element_type=jnp.float32)
                       + g_ref[...] + b_ref[...])

    return pl.pallas_call(
        body, grid=grid,
        in_specs=[xspec, wspec, w2spec, spec, bspec],
        out_specs=(spec, spec),
        out_shape=(jax.ShapeDtypeStruct((n, h), jnp.float32),) * 2,
        compiler_params=pltpu.CompilerParams(
            dimension_semantics=("parallel",)))(
            f_bonds, W_i, Wf1, gwb0, b_fusion.reshape(1, h))


def tc_step(msg1, W_h, Wf1, inp, g1, gwb_next, b_fusion):
    n, h = msg1.shape
    assert n % BN == 0
    grid = (n // BN,)
    spec = pl.BlockSpec((BN, h), lambda i: (i, 0))
    wspec = pl.BlockSpec((h, h), lambda i: (0, 0))
    bspec = pl.BlockSpec((1, h), lambda i: (0, 0))

    def body(x_ref, wh_ref, wf_ref, inp_ref, g1_ref, g_ref, b_ref, o_ref):
        m = jnp.dot(x_ref[...], wh_ref[...],
                    preferred_element_type=jnp.float32)
        nxt = jnp.maximum(inp_ref[...] + g1_ref[...] - _pairswap(m), 0.0)
        o_ref[...] = (jnp.dot(nxt, wf_ref[...],
                              preferred_element_type=jnp.float32)
                      + g_ref[...] + b_ref[...])

    return pl.pallas_call(
        body, grid=grid,
        in_specs=[spec, wspec, wspec, spec, spec, spec, bspec],
        out_specs=spec,
        out_shape=jax.ShapeDtypeStruct((n, h), jnp.float32),
        compiler_params=pltpu.CompilerParams(
            dimension_semantics=("parallel",)))(
            msg1, W_h, Wf1, inp, g1, gwb_next, b_fusion.reshape(1, h))


def tc_combine(x, w, inp, g1):
    n, h = x.shape
    assert n % BN == 0
    grid = (n // BN,)
    spec = pl.BlockSpec((BN, h), lambda i: (i, 0))
    wspec = pl.BlockSpec((h, h), lambda i: (0, 0))

    def body(x_ref, w_ref, inp_ref, g1_ref, o_ref):
        m = jnp.dot(x_ref[...], w_ref[...], preferred_element_type=jnp.float32)
        o_ref[...] = jnp.maximum(inp_ref[...] + g1_ref[...] - _pairswap(m), 0.0)

    return pl.pallas_call(
        body, grid=grid, in_specs=[spec, wspec, spec, spec], out_specs=spec,
        out_shape=jax.ShapeDtypeStruct((n, h), jnp.float32),
        compiler_params=pltpu.CompilerParams(
            dimension_semantics=("parallel",)))(x, w, inp, g1)


def _sc_mesh():
    return plsc.VectorSubcoreMesh(core_axis_name="c", subcore_axis_name="s")


def _wid():
    return lax.axis_index("s") * NC + lax.axis_index("c")


def sc_gather_rows(table, idx, replicate=False):
    v, hc = table.shape
    s = 2 if hc <= 128 else 1
    ch = s * 128
    sc_rows = 1024
    rounds = sc_rows // ch
    b = idx.shape[0]
    bp = _cdivmul(b, NW * sc_rows)
    if bp != b:
        pad = jnp.arange(bp - b, dtype=jnp.int32) % v
        idx = jnp.concatenate([idx, pad])
    if replicate:
        table = jnp.tile(table, (NW, 1))
    idx2 = idx.reshape(bp // 128, 128)
    bpw = bp // NW
    nch = bpw // sc_rows

    @functools.partial(
        pl.kernel,
        out_type=jax.ShapeDtypeStruct((bp, hc), jnp.float32),
        mesh=_sc_mesh(),
        scratch_types=[pltpu.VMEM((8, 128), jnp.int32),
                       pltpu.VMEM((ch, hc), jnp.float32),
                       pltpu.VMEM((ch, hc), jnp.float32),
                       pltpu.SemaphoreType.DMA,
                       pltpu.SemaphoreType.DMA],
    )
    def k(table_h, idx_h, out_h, idx_v, rows_a, rows_b, isem, gsem):
        base = _wid() * bpw

        def idx_cp(i):
            off = pl.multiple_of(base + i * sc_rows, 1024)
            return pltpu.make_async_copy(
                idx_h.at[pl.ds(pl.multiple_of(off // 128, 8), 8)], idx_v, isem)

        def fire(buf, rr):
            return [pltpu.async_copy(table_h.at[idx_v.at[rr * s + j]],
                                     buf.at[pl.ds(j * 128, 128)], gsem)
                    for j in range(s)]

        idx_cp(0).start()

        def chunk(i, c):
            off = pl.multiple_of(base + i * sc_rows, 1024)
            idx_cp(i).wait()
            if replicate:
                woff = _wid() * v
                for r8 in range(8):
                    for kk in range(8):
                        sl = pl.ds(kk * 16, 16)
                        idx_v[r8, sl] = idx_v[r8, sl] + woff
            cps = fire(rows_a, 0)
            for rr in range(rounds):
                buf = rows_a if rr % 2 == 0 else rows_b
                for cp in cps:
                    cp.wait()
                if rr + 1 < rounds:
                    cps = fire(rows_b if rr % 2 == 0 else rows_a, rr + 1)
                pltpu.sync_copy(
                    buf,
                    out_h.at[pl.ds(pl.multiple_of(off + rr * ch, ch), ch)])

            @pl.when(i + 1 < nch)
            def _():
                idx_cp(i + 1).start()

            return c

        lax.fori_loop(0, nch, chunk, 0)

    return k(table, idx2)


def sc_gather_sum(table, idx2d, nap):
    na, nb = idx2d.shape
    v, h = table.shape
    s = 2 if h <= 128 else 1
    ch = s * 128
    arh = ch // nb
    asc = 1024 // nb
    rounds = 1024 // ch
    assert nap % (NW * asc) == 0
    idx = idx2d
    if nap != na:
        idx = jnp.pad(idx, ((0, nap - na), (0, 0)))
    idxf = idx.reshape(nap * nb // 128, 128)
    apw = nap // NW
    nch = apw // asc
    hb = h // 16

    @functools.partial(
        pl.kernel,
        out_type=jax.ShapeDtypeStruct((nap, h), jnp.float32),
        mesh=_sc_mesh(),
        scratch_types=[pltpu.VMEM((8, 128), jnp.int32),
                       pltpu.VMEM((ch, h), jnp.float32),
                       pltpu.VMEM((ch, h), jnp.float32),
                       pltpu.VMEM((asc, h), jnp.float32),
                       pltpu.SemaphoreType.DMA,
                       pltpu.SemaphoreType.DMA],
    )
    def k(table_h, idx_h, out_h, idx_v, rows_a, rows_b, out_v, isem, gsem):
        base = _wid() * apw

        def idx_cp(i):
            aoff = pl.multiple_of(base + i * asc, asc)
            return pltpu.make_async_copy(
                idx_h.at[pl.ds(pl.multiple_of(aoff * nb // 128, 8), 8)],
                idx_v, isem)

        def fire(buf, rr):
            for j in range(s):
                pltpu.make_async_copy(table_h.at[idx_v.at[rr * s + j]],
                                      buf.at[pl.ds(j * 128, 128)],
                                      gsem).start()

        def drain(buf):
            for j in range(s):
                pltpu.make_async_copy(table_h.at[idx_v.at[0]],
                                      buf.at[pl.ds(j * 128, 128)],
                                      gsem).wait()

        def consume(buf, rr):
            def atom(a, c2):
                r0 = a * nb
                for hh in range(hb):
                    sl = pl.ds(hh * 16, 16)
                    acc = buf[r0, sl]
                    for j in range(1, nb):
                        acc = acc + buf[r0 + j, sl]
                    out_v[rr * arh + a, sl] = acc
                return c2

            lax.fori_loop(0, arh, atom, 0)

        idx_cp(0).start()

        def chunk(i, c):
            aoff = pl.multiple_of(base + i * asc, asc)
            idx_cp(i).wait()
            fire(rows_a, 0)

            def rpair(rp, c2):
                rr0 = 2 * rp
                drain(rows_a)
                fire(rows_b, rr0 + 1)
                consume(rows_a, rr0)
                drain(rows_b)

                @pl.when(rr0 + 2 < rounds)
                def _():
                    fire(rows_a, rr0 + 2)

                consume(rows_b, rr0 + 1)
                return c2

            lax.fori_loop(0, rounds // 2, rpair, 0)
            pltpu.sync_copy(out_v,
                            out_h.at[pl.ds(pl.multiple_of(aoff, asc), asc)])

            @pl.when(i + 1 < nch)
            def _():
                idx_cp(i + 1).start()

            return c

        lax.fori_loop(0, nch, chunk, 0)

    return k(table, idxf)


def sc_segsum(x, seg, nsegp):
    npts, h = x.shape
    assert npts % (NW * 128) == 0
    apw = npts // NW
    nch = apw // 128
    zeros = jnp.zeros((nsegp, h), jnp.float32)

    @functools.partial(
        pl.kernel,
        out_type=jax.ShapeDtypeStruct((NC, nsegp, h), jnp.float32),
        mesh=_sc_mesh(),
        scratch_types=[pltpu.VMEM((128,), jnp.int32),
                       pltpu.VMEM((128, h), jnp.float32),
                       pltpu.VMEM_SHARED((nsegp, h), jnp.float32)],
    )
    def k(x_h, seg_h, z_h, out_h, seg_v, x_v, acc_sh):
        sid = lax.axis_index("s")
        cid = lax.axis_index("c")
        base = _wid() * apw

        @pl.when(sid == 0)
        def _():
            pltpu.sync_copy(z_h, acc_sh)

        plsc.subcore_barrier()

        def chunk(i, c):
            off = pl.multiple_of(base + i * 128, 128)
            pltpu.sync_copy(seg_h.at[pl.ds(off, 128)], seg_v)
            pltpu.sync_copy(x_h.at[pl.ds(off, 128)], x_v)
            pltpu.sync_copy(x_v, acc_sh.at[seg_v], add=True)
            return c

        lax.fori_loop(0, nch, chunk, 0)
        plsc.subcore_barrier()

        @pl.when(sid == 0)
        def _():
            pltpu.sync_copy(acc_sh, out_h.at[cid])

    return k(x, seg, zeros)


def _segmean(x, seg, n, npad, nsegp, counts):
    xp = jnp.pad(x, ((0, npad - x.shape[0]), (0, 0)))
    sp = jnp.pad(seg.astype(jnp.int32), (0, npad - seg.shape[0]))
    parts = sc_segsum(xp, sp, nsegp)
    sums = parts[0, :n] + parts[1, :n]
    return jnp.where(counts[:, None] > 0,
                     sums / jnp.maximum(counts, 1.0)[:, None], 0.0)


@jax.jit
def _run(f_atoms, f_bonds, a2b, b2a, b2revb, atom_seg,
         f_frags_atoms, f_frags_bonds, frags_a2b, frags_b2a, frags_b2revb,
         frags_atom_seg, a2frag, W_i, W_h, W_fusion, b_fusion, W_o, b_o):
    H = W_h.shape[0]
    NA, MAXNB = a2b.shape
    NB_ = b2a.shape[0]
    FNA, FMAXNB = frags_a2b.shape
    NAP = _cdivmul(NA, 1024)
    FNAP = _cdivmul(FNA, 2048)
    NSP = _cdivmul(NA, NW * 128)
    FNSP = _cdivmul(FNA, NW * 128)
    NSEGP = _cdivmul(N_MOLS, 8)

    Wf1, Wf2 = W_fusion[:H], W_fusion[H:]

    a2b = a2b.astype(jnp.int32)
    b2a = b2a.astype(jnp.int32)
    frags_a2b = frags_a2b.astype(jnp.int32)
    frags_b2a = frags_b2a.astype(jnp.int32)
    a2frag = a2frag.astype(jnp.int32)
    counts = jax.ops.segment_sum(jnp.ones((NA,), jnp.float32),
                                 atom_seg, num_segments=N_MOLS)
    fcounts = jax.ops.segment_sum(jnp.ones((FNA,), jnp.float32),
                                  frags_atom_seg, num_segments=N_FRAG_MOLS)

    frags_input, fb = tc_matmul(f_frags_bonds, W_i, both=True)
    ffs = []
    for _ in range(DEPTH - 1):
        fA = sc_gather_sum(fb, frags_a2b, FNAP)
        fAh = tc_matmul(fA, W_h)
        Df = sc_gather_rows(fAh, frags_b2a)
        fb = tc_combine(fb, W_h, frags_input, Df)
        fA2 = sc_gather_sum(fb, frags_a2b, FNAP)
        a_in = jnp.concatenate([f_frags_atoms, fA2[:FNA]], axis=1)
        fh = tc_matmul(a_in, W_o, b=b_o, relu=True)
        ffm = _segmean(fh, frags_atom_seg, N_FRAG_MOLS, FNSP, NSEGP, fcounts)
        ffs.append(jnp.concatenate([jnp.zeros((1, H), jnp.float32), ffm], 0))

    gwbs = []
    for t in range(DEPTH - 1):
        gwtab = tc_matmul(ffs[t], Wf2)
        gat = sc_gather_rows(gwtab, a2frag, replicate=True)
        gwbs.append(sc_gather_rows(gat, b2a))

    inp, msg1 = tc_init(f_bonds, W_i, Wf1, gwbs[0], b_fusion)
    for t in range(DEPTH - 1):
        A = sc_gather_sum(msg1, a2b, NAP)
        C = tc_matmul(A, W_h)
        G1 = sc_gather_rows(C, b2a)
        if t + 1 < DEPTH - 1:
            msg1 = tc_step(msg1, W_h, Wf1, inp, G1, gwbs[t + 1], b_fusion)
        else:
            message = tc_combine(msg1, W_h, inp, G1)

    A2 = sc_gather_sum(message, a2b, NAP)
    a_in = jnp.concatenate([f_atoms, A2[:NA]], axis=1)
    atom_hiddens = tc_matmul(a_in, W_o, b=b_o, relu=True)
    mol_vecs = _segmean(atom_hiddens, atom_seg, N_MOLS, NSP, NSEGP, counts)
    return mol_vecs, atom_hiddens, ffs[-1]


def kernel(f_atoms, f_bonds, a2b, b2a, b2revb, atom_seg, f_frags_atoms,
           f_frags_bonds, frags_a2b, frags_b2a, frags_b2revb, frags_atom_seg,
           a2frag, W_i, W_h, W_fusion, b_fusion, W_o, b_o):
    return _run(f_atoms, f_bonds, a2b, b2a, b2revb, atom_seg, f_frags_atoms,
                f_frags_bonds, frags_a2b, frags_b2a, frags_b2revb,
                frags_atom_seg, a2frag, W_i, W_h, W_fusion, b_fusion, W_o, b_o)
